# baseline (speedup 1.0000x reference)
import functools

import jax
import jax.numpy as jnp
from jax import lax
from jax.experimental import pallas as pl
from jax.experimental.pallas import tpu as pltpu
from jax.experimental.pallas import tpu_sc as plsc

N = 10000
E = 320000
D_IN = 128
D_EDGE = 16
HID = 16
HEADS = 8
OUT = 128
MLM = 9

NC = 2
NS = 16
NTILES = NC * NS
CHUNK = 48
NCHUNK = 210
EPT = CHUNK * NCHUNK
E_PAD = EPT * NTILES
NPAD = 10240
SINK = NPAD - 1
ROWS_PER_TILE = NPAD // NS
DEN_R = NPAD // 8


def _sc_edge_pass(heads):
    mesh = plsc.VectorSubcoreMesh(
        core_axis_name="c", subcore_axis_name="s", num_cores=NC, num_subcores=NS)

    @functools.partial(
        pl.kernel,
        out_type=(
            jax.ShapeDtypeStruct((NC, NPAD, 128), jnp.float32),
            jax.ShapeDtypeStruct((NC, DEN_R, 128), jnp.float32),
        ),
        mesh=mesh,
        scratch_types=(
            [pltpu.VMEM((CHUNK,), jnp.int32)] * 6
            + [pltpu.VMEM((CHUNK, 128), jnp.float32)] * 4
            + [pltpu.VMEM((128,), jnp.float32)]
            + [
                pltpu.VMEM_SHARED((NPAD, 128), jnp.float32),
                pltpu.VMEM_SHARED((DEN_R, 128), jnp.float32),
            ]
            + [pltpu.SemaphoreType.DMA] * 8
        ),
    )
    def body(src_hbm, dst_hbm, xl_hbm, xr_hbm, e_hbm, att_hbm, zeros_hbm,
             out_hbm, outden_hbm, src_a, dst_a, q2_a, src_b, dst_b, q2_b,
             xj_a, e_a, xj_b, e_b, att_v, acc_sh, den_sh,
             ga1, ga2, gb1, gb2, sa1, sa2, sb1, sb2):
        cid = lax.axis_index("c")
        sid = lax.axis_index("s")
        tid = cid * NS + sid
        sets = (
            (src_a, dst_a, q2_a, xj_a, e_a, ga1, ga2, sa1, sa2),
            (src_b, dst_b, q2_b, xj_b, e_b, gb1, gb2, sb1, sb2),
        )

        pltpu.sync_copy(zeros_hbm, acc_sh.at[pl.ds(sid * ROWS_PER_TILE, ROWS_PER_TILE)])
        pltpu.sync_copy(zeros_hbm.at[pl.ds(0, DEN_R // NS)],
                        den_sh.at[pl.ds(sid * (DEN_R // NS), DEN_R // NS)])
        pltpu.sync_copy(att_hbm, att_v)
        plsc.subcore_barrier()

        lane = lax.iota(jnp.int32, 16)
        lane_half = lax.shift_right_logical(lane, 3)
        lane7 = lane & 7
        att_regs = [att_v[pl.ds(16 * h, 16)] for h in range(8)]
        zero16 = jnp.zeros((16,), jnp.float32)
        zero16i = jnp.zeros((16,), jnp.int32)
        eight16i = jnp.full((16,), 8, jnp.int32)

        def splat_sum(v):
            for k in (8, 4, 2, 1):
                v = v + v.at[lane ^ k].get(mode="promise_in_bounds")
            return v

        def make_compute(bufset):
            src_v, dst_v, q2_v, xj_v, e_v = bufset[:5]

            def den_store(i, dstg, j, ex_row):
                dsp = dstg.at[jnp.full((16,), j, jnp.int32)].get(mode="promise_in_bounds")
                ex16 = ex_row.at[lane7].get(mode="promise_in_bounds")
                dq = dsp & 7
                for w in range(4):
                    e_v[i, pl.ds(16 * w, 16)] = jnp.where(
                        lane_half + 2 * w == dq, ex16, zero16)

            def edge8(i, dstg, j):
                ex_row = zero16
                for t in range(4):
                    ha, hb = 2 * t, 2 * t + 1
                    sla = pl.ds(ha * 16, 16)
                    slb = pl.ds(hb * 16, 16)
                    xja = xj_v[i, sla]
                    xjb = xj_v[i, slb]
                    ma = xja + e_v[i, sla]
                    mb = xjb + e_v[i, slb]
                    ma = jnp.maximum(ma, ma * 0.2)
                    mb = jnp.maximum(mb, mb * 0.2)
                    pa = ma * att_regs[ha]
                    pb = mb * att_regs[hb]
                    fa = pa + pa.at[lane ^ 8].get(mode="promise_in_bounds")
                    fb = pb + pb.at[lane ^ 8].get(mode="promise_in_bounds")
                    tv = jnp.where(lane_half == 0, fa, fb)
                    for k in (4, 2, 1):
                        tv = tv + tv.at[lane ^ k].get(mode="promise_in_bounds")
                    ext = jnp.exp(tv)
                    exa = ext.at[zero16i].get(mode="promise_in_bounds")
                    exb = ext.at[eight16i].get(mode="promise_in_bounds")
                    xj_v[i, sla] = xja * exa
                    xj_v[i, slb] = xjb * exb
                    ex_row = jnp.where(lane == ha, exa, ex_row)
                    ex_row = jnp.where(lane == hb, exb, ex_row)
                den_store(i, dstg, j, ex_row)

            def edge1(i, dstg, j):
                acc = zero16
                xjs = []
                for h in range(8):
                    sl = pl.ds(h * 16, 16)
                    xj = xj_v[i, sl]
                    xjs.append(xj)
                    m = xj + e_v[i, sl]
                    m = jnp.maximum(m, m * 0.2)
                    acc = acc + m * att_regs[h]
                exh = jnp.exp(splat_sum(acc))
                for h in range(8):
                    xj_v[i, pl.ds(h * 16, 16)] = xjs[h] * exh
                ex_row = jnp.where(lane == 0, exh, zero16)
                den_store(i, dstg, j, ex_row)

            edge = edge8 if heads == 8 else edge1

            def group_body(g, carry):
                dstg = dst_v[pl.ds(g * 16, 16)]
                q2_v[pl.ds(g * 16, 16)] = lax.shift_right_logical(dstg, 3)

                def octet(j8, c):
                    for dj in range(8):
                        j = j8 * 8 + dj
                        edge(g * 16 + j, dstg, j)
                    return c

                lax.fori_loop(0, 2, octet, 0)
                return carry

            return group_body

        computes = [make_compute(s) for s in sets]

        def issue_gathers(k, bufset):
            src_v, dst_v, _, xj_v, e_v, g1, g2 = bufset[:7]
            base = tid * EPT + k * CHUNK
            pltpu.sync_copy(src_hbm.at[pl.ds(base, CHUNK)], src_v)
            pltpu.sync_copy(dst_hbm.at[pl.ds(base, CHUNK)], dst_v)
            pltpu.sync_copy(e_hbm.at[pl.ds(base, CHUNK)], e_v)
            cp1 = pltpu.async_copy(xl_hbm.at[src_v], xj_v, g1)
            cp2 = pltpu.make_async_copy(xr_hbm.at[dst_v], e_v, g2)
            cp2.start(add=True)
            return cp1, cp2

        def start_scatters(bufset):
            _, dst_v, q2_v, xj_v, e_v, _, _, s1, s2 = bufset
            c1 = pltpu.make_async_copy(xj_v, acc_sh.at[dst_v], s1)
            c1.start(add=True)
            c2 = pltpu.make_async_copy(e_v, den_sh.at[q2_v], s2)
            c2.start(add=True)

        def wait_scatters(bufset):
            _, dst_v, q2_v, xj_v, e_v, _, _, s1, s2 = bufset
            pltpu.make_async_copy(xj_v, acc_sh.at[dst_v], s1).wait()
            pltpu.make_async_copy(e_v, den_sh.at[q2_v], s2).wait()

        def pair_body(k2, carry):
            k = 2 * k2

            @pl.when(k2 > 0)
            def _():
                wait_scatters(sets[0])
                wait_scatters(sets[1])

            cpa = issue_gathers(k, sets[0])
            cpb = issue_gathers(k + 1, sets[1])
            for cp in cpa:
                cp.wait()
            lax.fori_loop(0, CHUNK // 16, computes[0], 0)
            start_scatters(sets[0])
            for cp in cpb:
                cp.wait()
            lax.fori_loop(0, CHUNK // 16, computes[1], 0)
            start_scatters(sets[1])
            return carry

        lax.fori_loop(0, NCHUNK // 2, pair_body, 0)
        wait_scatters(sets[0])
        wait_scatters(sets[1])

        plsc.subcore_barrier()
        pltpu.sync_copy(
            acc_sh.at[pl.ds(sid * ROWS_PER_TILE, ROWS_PER_TILE)],
            out_hbm.at[cid, pl.ds(sid * ROWS_PER_TILE, ROWS_PER_TILE)])
        pltpu.sync_copy(
            den_sh.at[pl.ds(sid * (DEN_R // NS), DEN_R // NS)],
            outden_hbm.at[cid, pl.ds(sid * (DEN_R // NS), DEN_R // NS)])

    return body


_sc_edge8 = _sc_edge_pass(8)
_sc_edge1 = _sc_edge_pass(1)


def _edge_proj_body(ea_ref, w1_ref, w2_ref, w3_ref, o1_ref, o2_ref, o3_ref):
    ea = ea_ref[...]
    o1_ref[...] = ea @ w1_ref[...]
    o2_ref[...] = ea @ w2_ref[...]
    o3_ref[...] = ea @ w3_ref[...]


def _edge_proj(edge_attr, We1, We2, We3):
    BE = 8064
    sh = jax.ShapeDtypeStruct((E_PAD, 128), jnp.float32)
    return pl.pallas_call(
        _edge_proj_body,
        out_shape=(sh, sh, sh),
        grid=(E_PAD // BE,),
        in_specs=[
            pl.BlockSpec((BE, D_EDGE), lambda i: (i, 0)),
            pl.BlockSpec((D_EDGE, 128), lambda i: (0, 0)),
            pl.BlockSpec((D_EDGE, 128), lambda i: (0, 0)),
            pl.BlockSpec((D_EDGE, 128), lambda i: (0, 0)),
        ],
        out_specs=(
            pl.BlockSpec((BE, 128), lambda i: (i, 0)),
            pl.BlockSpec((BE, 128), lambda i: (i, 0)),
            pl.BlockSpec((BE, 128), lambda i: (i, 0)),
        ),
    )(edge_attr, We1, We2, We3)


def _node_proj_body(x_ref, wl_ref, wr_ref, xl_ref, xr_ref):
    x = x_ref[...]
    xl_ref[...] = x @ wl_ref[...]
    xr_ref[...] = x @ wr_ref[...]


def _node_proj1(x, Wl, Wr):
    BN = 2048
    sh = jax.ShapeDtypeStruct((NPAD, 128), jnp.float32)
    return pl.pallas_call(
        _node_proj_body,
        out_shape=(sh, sh),
        grid=(NPAD // BN,),
        in_specs=[
            pl.BlockSpec((BN, 128), lambda i: (i, 0)),
            pl.BlockSpec((128, 128), lambda i: (0, 0)),
            pl.BlockSpec((128, 128), lambda i: (0, 0)),
        ],
        out_specs=(
            pl.BlockSpec((BN, 128), lambda i: (i, 0)),
            pl.BlockSpec((BN, 128), lambda i: (i, 0)),
        ),
    )(x, Wl, Wr)


def _combine(acc, den, bias, Wl, Wr, ch):
    BN = 2048

    def body(acc_ref, den_ref, b_ref, wl_ref, wr_ref, xl_ref, xr_ref):
        msg = acc_ref[0] + acc_ref[1]
        den_blk = den_ref[0] + den_ref[1]
        r = (lax.broadcasted_iota(jnp.int32, (8, 128), 1) // ch
             == lax.broadcasted_iota(jnp.int32, (8, 128), 0)).astype(jnp.float32)
        denb = den_blk @ r
        h = msg / (denb + 1e-16) + b_ref[...][None, :]
        h = jnp.maximum(h, 0.0)
        xl_ref[...] = h @ wl_ref[...]
        xr_ref[...] = h @ wr_ref[...]

    sh = jax.ShapeDtypeStruct((NPAD, 128), jnp.float32)
    return pl.pallas_call(
        body,
        out_shape=(sh, sh),
        grid=(NPAD // BN,),
        in_specs=[
            pl.BlockSpec((NC, BN, 128), lambda i: (0, i, 0)),
            pl.BlockSpec((NC, BN, 8), lambda i: (0, i, 0)),
            pl.BlockSpec((128,), lambda i: (0,)),
            pl.BlockSpec((128, 128), lambda i: (0, 0)),
            pl.BlockSpec((128, 128), lambda i: (0, 0)),
        ],
        out_specs=(
            pl.BlockSpec((BN, 128), lambda i: (i, 0)),
            pl.BlockSpec((BN, 128), lambda i: (i, 0)),
        ),
    )(acc, den, bias, Wl, Wr)


def _final(acc, den, b3, Wm1, bm1, Wm2, bm2):
    BN = 2000

    def body(acc_ref, den_ref, b_ref, wm1_ref, bm1_ref, wm2_ref, bm2_ref, o_ref):
        msg = acc_ref[0] + acc_ref[1]
        den_blk = den_ref[0] + den_ref[1]
        r = (lax.broadcasted_iota(jnp.int32, (8, 128), 0) == 0).astype(jnp.float32)
        denb = den_blk @ r
        h = msg / (denb + 1e-16) + b_ref[...][None, :]
        t = jnp.maximum(h @ wm1_ref[...] + bm1_ref[...][None, :], 0.0)
        o_ref[...] = t @ wm2_ref[...] + bm2_ref[...][None, :]

    return pl.pallas_call(
        body,
        out_shape=jax.ShapeDtypeStruct((N, MLM), jnp.float32),
        grid=(N // BN,),
        in_specs=[
            pl.BlockSpec((NC, BN, 128), lambda i: (0, i, 0)),
            pl.BlockSpec((NC, BN, 8), lambda i: (0, i, 0)),
            pl.BlockSpec((128,), lambda i: (0,)),
            pl.BlockSpec((128, 128), lambda i: (0, 0)),
            pl.BlockSpec((128,), lambda i: (0,)),
            pl.BlockSpec((128, MLM), lambda i: (0, 0)),
            pl.BlockSpec((MLM,), lambda i: (0,)),
        ],
        out_specs=pl.BlockSpec((BN, MLM), lambda i: (i, 0)),
    )(acc, den, b3, Wm1, bm1, Wm2, bm2)


def kernel(x, edge_index, edge_attr, batch, Wl1, Wr1, We1, att1, b1, Wl2, Wr2, We2, att2, b2, Wl3, Wr3, We3, att3, b3, Wm1, bm1, Wm2, bm2):
    npad_e = E_PAD - E
    src = jnp.concatenate(
        [edge_index[0].astype(jnp.int32), jnp.zeros((npad_e,), jnp.int32)])
    dst = jnp.concatenate(
        [edge_index[1].astype(jnp.int32), jnp.full((npad_e,), SINK, jnp.int32)])
    edge_attr = jnp.concatenate(
        [edge_attr, jnp.zeros((npad_e, D_EDGE), jnp.float32)])
    x = jnp.concatenate([x, jnp.zeros((NPAD - N, D_IN), jnp.float32)])
    zeros = jnp.zeros((ROWS_PER_TILE, 128), jnp.float32)
    att1f = att1.reshape(-1)
    att2f = att2.reshape(-1)
    att3f = att3.reshape(-1)

    def den_view(den):
        return den.reshape(NC, DEN_R, 16, 8)[:, :, :8, :].reshape(NC, NPAD, 8)

    e1, e2, e3 = _edge_proj(edge_attr, We1, We2, We3)
    xl1, xr1 = _node_proj1(x, Wl1, Wr1)
    acc1, den1 = _sc_edge8(src, dst, xl1, xr1, e1, att1f, zeros)
    xl2, xr2 = _combine(acc1, den_view(den1), b1, Wl2, Wr2, HID)
    acc2, den2 = _sc_edge8(src, dst, xl2, xr2, e2, att2f, zeros)
    xl3, xr3 = _combine(acc2, den_view(den2), b2, Wl3, Wr3, HID)
    acc3, den3 = _sc_edge1(src, dst, xl3, xr3, e3, att3f, zeros)
    return _final(acc3, den_view(den3), b3, Wm1, bm1, Wm2, bm2)


# speedup vs baseline: 1.0789x; 1.0789x over previous
import functools

import jax
import jax.numpy as jnp
from jax import lax
from jax.experimental import pallas as pl
from jax.experimental.pallas import tpu as pltpu
from jax.experimental.pallas import tpu_sc as plsc

N = 10000
E = 320000
D_IN = 128
D_EDGE = 16
HID = 16
HEADS = 8
OUT = 128
MLM = 9

NC = 2
NS = 16
NTILES = NC * NS
CHUNK = 48
NCHUNK = 210
EPT = CHUNK * NCHUNK
E_PAD = EPT * NTILES
NPAD = 10240
SINK = NPAD - 1
ROWS_PER_TILE = NPAD // NS
DEN_R = NPAD // 8


def _sc_edge_pass(heads):
    mesh = plsc.VectorSubcoreMesh(
        core_axis_name="c", subcore_axis_name="s", num_cores=NC, num_subcores=NS)

    @functools.partial(
        pl.kernel,
        out_type=(
            jax.ShapeDtypeStruct((NC, NPAD, 128), jnp.float32),
            jax.ShapeDtypeStruct((NC, DEN_R, 128), jnp.float32),
        ),
        mesh=mesh,
        scratch_types=(
            [pltpu.VMEM((CHUNK,), jnp.int32)] * 6
            + [pltpu.VMEM((CHUNK, 128), jnp.float32)] * 6
            + [pltpu.VMEM((128,), jnp.float32)]
            + [
                pltpu.VMEM_SHARED((NPAD, 128), jnp.float32),
                pltpu.VMEM_SHARED((DEN_R, 128), jnp.float32),
            ]
            + [pltpu.SemaphoreType.DMA] * 10
        ),
    )
    def body(src_hbm, dst_hbm, xl_hbm, xr_hbm, e_hbm, att_hbm, zeros_hbm,
             out_hbm, outden_hbm, src_a, dst_a, q2_a, src_b, dst_b, q2_b,
             xj_a, xi_a, e_a, xj_b, xi_b, e_b, att_v, acc_sh, den_sh,
             ga1, ga2, ga3, gb1, gb2, gb3, sa1, sa2, sb1, sb2):
        cid = lax.axis_index("c")
        sid = lax.axis_index("s")
        tid = cid * NS + sid
        sets = (
            (src_a, dst_a, q2_a, xj_a, xi_a, e_a, ga1, ga2, ga3, sa1, sa2),
            (src_b, dst_b, q2_b, xj_b, xi_b, e_b, gb1, gb2, gb3, sb1, sb2),
        )

        pltpu.sync_copy(zeros_hbm, acc_sh.at[pl.ds(sid * ROWS_PER_TILE, ROWS_PER_TILE)])
        pltpu.sync_copy(zeros_hbm.at[pl.ds(0, DEN_R // NS)],
                        den_sh.at[pl.ds(sid * (DEN_R // NS), DEN_R // NS)])
        pltpu.sync_copy(att_hbm, att_v)
        plsc.subcore_barrier()

        lane = lax.iota(jnp.int32, 16)
        lane_half = lax.shift_right_logical(lane, 3)
        lane7 = lane & 7
        att_regs = [att_v[pl.ds(16 * h, 16)] for h in range(8)]
        zero16 = jnp.zeros((16,), jnp.float32)
        zero16i = jnp.zeros((16,), jnp.int32)
        eight16i = jnp.full((16,), 8, jnp.int32)

        def splat_sum(v):
            for k in (8, 4, 2, 1):
                v = v + v.at[lane ^ k].get(mode="promise_in_bounds")
            return v

        def make_compute(bufset):
            src_v, dst_v, q2_v, xj_v, xi_v, e_v = bufset[:6]

            def den_store(i, dstg, j, ex_row):
                dsp = dstg.at[jnp.full((16,), j, jnp.int32)].get(mode="promise_in_bounds")
                ex16 = ex_row.at[lane7].get(mode="promise_in_bounds")
                dq = dsp & 7
                for w in range(4):
                    xi_v[i, pl.ds(16 * w, 16)] = jnp.where(
                        lane_half + 2 * w == dq, ex16, zero16)

            def edge8(i, dstg, j):
                ex_row = zero16
                for t in range(4):
                    ha, hb = 2 * t, 2 * t + 1
                    sla = pl.ds(ha * 16, 16)
                    slb = pl.ds(hb * 16, 16)
                    xja = xj_v[i, sla]
                    xjb = xj_v[i, slb]
                    ma = xi_v[i, sla] + xja + e_v[i, sla]
                    mb = xi_v[i, slb] + xjb + e_v[i, slb]
                    ma = jnp.maximum(ma, ma * 0.2)
                    mb = jnp.maximum(mb, mb * 0.2)
                    pa = ma * att_regs[ha]
                    pb = mb * att_regs[hb]
                    fa = pa + pa.at[lane ^ 8].get(mode="promise_in_bounds")
                    fb = pb + pb.at[lane ^ 8].get(mode="promise_in_bounds")
                    tv = jnp.where(lane_half == 0, fa, fb)
                    for k in (4, 2, 1):
                        tv = tv + tv.at[lane ^ k].get(mode="promise_in_bounds")
                    ext = jnp.exp(tv)
                    exa = ext.at[zero16i].get(mode="promise_in_bounds")
                    exb = ext.at[eight16i].get(mode="promise_in_bounds")
                    xj_v[i, sla] = xja * exa
                    xj_v[i, slb] = xjb * exb
                    ex_row = jnp.where(lane == ha, exa, ex_row)
                    ex_row = jnp.where(lane == hb, exb, ex_row)
                den_store(i, dstg, j, ex_row)

            def edge1(i, dstg, j):
                acc = zero16
                xjs = []
                for h in range(8):
                    sl = pl.ds(h * 16, 16)
                    xj = xj_v[i, sl]
                    xjs.append(xj)
                    m = xi_v[i, sl] + xj + e_v[i, sl]
                    m = jnp.maximum(m, m * 0.2)
                    acc = acc + m * att_regs[h]
                exh = jnp.exp(splat_sum(acc))
                for h in range(8):
                    xj_v[i, pl.ds(h * 16, 16)] = xjs[h] * exh
                ex_row = jnp.where(lane == 0, exh, zero16)
                den_store(i, dstg, j, ex_row)

            edge = edge8 if heads == 8 else edge1

            def group_body(g, carry):
                dstg = dst_v[pl.ds(g * 16, 16)]
                q2_v[pl.ds(g * 16, 16)] = lax.shift_right_logical(dstg, 3)

                def octet(j8, c):
                    for dj in range(8):
                        j = j8 * 8 + dj
                        edge(g * 16 + j, dstg, j)
                    return c

                lax.fori_loop(0, 2, octet, 0)
                return carry

            return group_body

        computes = [make_compute(s) for s in sets]

        def issue_gathers(k, bufset):
            src_v, dst_v, _, xj_v, xi_v, e_v, g1, g2, g3 = bufset[:9]
            base = tid * EPT + k * CHUNK
            pltpu.sync_copy(src_hbm.at[pl.ds(base, CHUNK)], src_v)
            pltpu.sync_copy(dst_hbm.at[pl.ds(base, CHUNK)], dst_v)
            cp1 = pltpu.async_copy(xl_hbm.at[src_v], xj_v, g1)
            cp2 = pltpu.async_copy(xr_hbm.at[dst_v], xi_v, g2)
            cp3 = pltpu.async_copy(e_hbm.at[pl.ds(base, CHUNK)], e_v, g3)
            return cp1, cp2, cp3

        def start_scatters(bufset):
            _, dst_v, q2_v, xj_v, xi_v, _, _, _, _, s1, s2 = bufset
            c1 = pltpu.make_async_copy(xj_v, acc_sh.at[dst_v], s1)
            c1.start(add=True)
            c2 = pltpu.make_async_copy(xi_v, den_sh.at[q2_v], s2)
            c2.start(add=True)

        def wait_scatters(bufset):
            _, dst_v, q2_v, xj_v, xi_v, _, _, _, _, s1, s2 = bufset
            pltpu.make_async_copy(xj_v, acc_sh.at[dst_v], s1).wait()
            pltpu.make_async_copy(xi_v, den_sh.at[q2_v], s2).wait()

        def pair_body(k2, carry):
            k = 2 * k2

            @pl.when(k2 > 0)
            def _():
                wait_scatters(sets[0])
                wait_scatters(sets[1])

            cpa = issue_gathers(k, sets[0])
            cpb = issue_gathers(k + 1, sets[1])
            for cp in cpa:
                cp.wait()
            lax.fori_loop(0, CHUNK // 16, computes[0], 0)
            start_scatters(sets[0])
            for cp in cpb:
                cp.wait()
            lax.fori_loop(0, CHUNK // 16, computes[1], 0)
            start_scatters(sets[1])
            return carry

        lax.fori_loop(0, NCHUNK // 2, pair_body, 0)
        wait_scatters(sets[0])
        wait_scatters(sets[1])

        plsc.subcore_barrier()
        pltpu.sync_copy(
            acc_sh.at[pl.ds(sid * ROWS_PER_TILE, ROWS_PER_TILE)],
            out_hbm.at[cid, pl.ds(sid * ROWS_PER_TILE, ROWS_PER_TILE)])
        pltpu.sync_copy(
            den_sh.at[pl.ds(sid * (DEN_R // NS), DEN_R // NS)],
            outden_hbm.at[cid, pl.ds(sid * (DEN_R // NS), DEN_R // NS)])

    return body


_sc_edge8 = _sc_edge_pass(8)
_sc_edge1 = _sc_edge_pass(1)


def _edge_proj_body(ea_ref, w1_ref, w2_ref, w3_ref, o1_ref, o2_ref, o3_ref):
    ea = ea_ref[...]
    o1_ref[...] = ea @ w1_ref[...]
    o2_ref[...] = ea @ w2_ref[...]
    o3_ref[...] = ea @ w3_ref[...]


def _edge_proj(edge_attr, We1, We2, We3):
    BE = 8064
    sh = jax.ShapeDtypeStruct((E_PAD, 128), jnp.float32)
    return pl.pallas_call(
        _edge_proj_body,
        out_shape=(sh, sh, sh),
        grid=(E_PAD // BE,),
        in_specs=[
            pl.BlockSpec((BE, D_EDGE), lambda i: (i, 0)),
            pl.BlockSpec((D_EDGE, 128), lambda i: (0, 0)),
            pl.BlockSpec((D_EDGE, 128), lambda i: (0, 0)),
            pl.BlockSpec((D_EDGE, 128), lambda i: (0, 0)),
        ],
        out_specs=(
            pl.BlockSpec((BE, 128), lambda i: (i, 0)),
            pl.BlockSpec((BE, 128), lambda i: (i, 0)),
            pl.BlockSpec((BE, 128), lambda i: (i, 0)),
        ),
    )(edge_attr, We1, We2, We3)


def _node_proj_body(x_ref, wl_ref, wr_ref, xl_ref, xr_ref):
    x = x_ref[...]
    xl_ref[...] = x @ wl_ref[...]
    xr_ref[...] = x @ wr_ref[...]


def _node_proj1(x, Wl, Wr):
    BN = 2048
    sh = jax.ShapeDtypeStruct((NPAD, 128), jnp.float32)
    return pl.pallas_call(
        _node_proj_body,
        out_shape=(sh, sh),
        grid=(NPAD // BN,),
        in_specs=[
            pl.BlockSpec((BN, 128), lambda i: (i, 0)),
            pl.BlockSpec((128, 128), lambda i: (0, 0)),
            pl.BlockSpec((128, 128), lambda i: (0, 0)),
        ],
        out_specs=(
            pl.BlockSpec((BN, 128), lambda i: (i, 0)),
            pl.BlockSpec((BN, 128), lambda i: (i, 0)),
        ),
    )(x, Wl, Wr)


def _combine(acc, den, bias, Wl, Wr, ch):
    BN = 2048

    def body(acc_ref, den_ref, b_ref, wl_ref, wr_ref, xl_ref, xr_ref):
        msg = acc_ref[0] + acc_ref[1]
        den_blk = den_ref[0] + den_ref[1]
        r = (lax.broadcasted_iota(jnp.int32, (8, 128), 1) // ch
             == lax.broadcasted_iota(jnp.int32, (8, 128), 0)).astype(jnp.float32)
        denb = den_blk @ r
        h = msg / (denb + 1e-16) + b_ref[...][None, :]
        h = jnp.maximum(h, 0.0)
        xl_ref[...] = h @ wl_ref[...]
        xr_ref[...] = h @ wr_ref[...]

    sh = jax.ShapeDtypeStruct((NPAD, 128), jnp.float32)
    return pl.pallas_call(
        body,
        out_shape=(sh, sh),
        grid=(NPAD // BN,),
        in_specs=[
            pl.BlockSpec((NC, BN, 128), lambda i: (0, i, 0)),
            pl.BlockSpec((NC, BN, 8), lambda i: (0, i, 0)),
            pl.BlockSpec((128,), lambda i: (0,)),
            pl.BlockSpec((128, 128), lambda i: (0, 0)),
            pl.BlockSpec((128, 128), lambda i: (0, 0)),
        ],
        out_specs=(
            pl.BlockSpec((BN, 128), lambda i: (i, 0)),
            pl.BlockSpec((BN, 128), lambda i: (i, 0)),
        ),
    )(acc, den, bias, Wl, Wr)


def _final(acc, den, b3, Wm1, bm1, Wm2, bm2):
    BN = 2000

    def body(acc_ref, den_ref, b_ref, wm1_ref, bm1_ref, wm2_ref, bm2_ref, o_ref):
        msg = acc_ref[0] + acc_ref[1]
        den_blk = den_ref[0] + den_ref[1]
        r = (lax.broadcasted_iota(jnp.int32, (8, 128), 0) == 0).astype(jnp.float32)
        denb = den_blk @ r
        h = msg / (denb + 1e-16) + b_ref[...][None, :]
        t = jnp.maximum(h @ wm1_ref[...] + bm1_ref[...][None, :], 0.0)
        o_ref[...] = t @ wm2_ref[...] + bm2_ref[...][None, :]

    return pl.pallas_call(
        body,
        out_shape=jax.ShapeDtypeStruct((N, MLM), jnp.float32),
        grid=(N // BN,),
        in_specs=[
            pl.BlockSpec((NC, BN, 128), lambda i: (0, i, 0)),
            pl.BlockSpec((NC, BN, 8), lambda i: (0, i, 0)),
            pl.BlockSpec((128,), lambda i: (0,)),
            pl.BlockSpec((128, 128), lambda i: (0, 0)),
            pl.BlockSpec((128,), lambda i: (0,)),
            pl.BlockSpec((128, MLM), lambda i: (0, 0)),
            pl.BlockSpec((MLM,), lambda i: (0,)),
        ],
        out_specs=pl.BlockSpec((BN, MLM), lambda i: (i, 0)),
    )(acc, den, b3, Wm1, bm1, Wm2, bm2)


def kernel(x, edge_index, edge_attr, batch, Wl1, Wr1, We1, att1, b1, Wl2, Wr2, We2, att2, b2, Wl3, Wr3, We3, att3, b3, Wm1, bm1, Wm2, bm2):
    npad_e = E_PAD - E
    src = jnp.concatenate(
        [edge_index[0].astype(jnp.int32), jnp.zeros((npad_e,), jnp.int32)])
    dst = jnp.concatenate(
        [edge_index[1].astype(jnp.int32), jnp.full((npad_e,), SINK, jnp.int32)])
    edge_attr = jnp.concatenate(
        [edge_attr, jnp.zeros((npad_e, D_EDGE), jnp.float32)])
    x = jnp.concatenate([x, jnp.zeros((NPAD - N, D_IN), jnp.float32)])
    zeros = jnp.zeros((ROWS_PER_TILE, 128), jnp.float32)
    att1f = att1.reshape(-1)
    att2f = att2.reshape(-1)
    att3f = att3.reshape(-1)

    def den_view(den):
        return den.reshape(NC, DEN_R, 16, 8)[:, :, :8, :].reshape(NC, NPAD, 8)

    e1, e2, e3 = _edge_proj(edge_attr, We1, We2, We3)
    xl1, xr1 = _node_proj1(x, Wl1, Wr1)
    acc1, den1 = _sc_edge8(src, dst, xl1, xr1, e1, att1f, zeros)
    xl2, xr2 = _combine(acc1, den_view(den1), b1, Wl2, Wr2, HID)
    acc2, den2 = _sc_edge8(src, dst, xl2, xr2, e2, att2f, zeros)
    xl3, xr3 = _combine(acc2, den_view(den2), b2, Wl3, Wr3, HID)
    acc3, den3 = _sc_edge1(src, dst, xl3, xr3, e3, att3f, zeros)
    return _final(acc3, den_view(den3), b3, Wm1, bm1, Wm2, bm2)
